# baseline (speedup 1.0000x reference)
import jax
import jax.numpy as jnp
from jax.experimental import pallas as pl
from jax.experimental.pallas import tpu as pltpu


def _x_body(p1_ref, p2_ref, x_ref):
    x1 = p1_ref[0, :, 0:1]
    y1 = p1_ref[0, :, 1:2]
    x2 = p2_ref[0, :, 0:1]
    y2 = p2_ref[0, :, 1:2]
    ones = jnp.ones_like(x1)
    x_ref[0] = jnp.concatenate(
        [x2 * x1, x2 * y1, x2, y2 * x1, y2 * y1, y2, x1, y1, ones], axis=1)


def _build_X(p1, p2):
    B, N, _ = p1.shape
    return pl.pallas_call(
        _x_body,
        grid=(B,),
        in_specs=[pl.BlockSpec((1, N, 2), lambda b: (b, 0, 0)),
                  pl.BlockSpec((1, N, 2), lambda b: (b, 0, 0))],
        out_specs=pl.BlockSpec((1, N, 9), lambda b: (b, 0, 0)),
        out_shape=jax.ShapeDtypeStruct((B, N, 9), jnp.float32),
        compiler_params=pltpu.CompilerParams(
            dimension_semantics=("parallel",)),
    )(p1, p2)


def _x4_body(p1_ref, p2_ref, pr2_ref, out_ref):
    x1 = p1_ref[0, :, 0:1]
    y1 = p1_ref[0, :, 1:2]
    x2 = p2_ref[0, :, 0:1]
    y2 = p2_ref[0, :, 1:2]
    p2r0 = pr2_ref[0, 0:1, :]
    p2r1 = pr2_ref[0, 1:2, :]
    p2r2 = pr2_ref[0, 2:3, :]
    ones = jnp.ones_like(x1)
    zero = jnp.zeros_like(x1)
    r0 = jnp.concatenate([-ones, zero, x1, zero], axis=1)
    r1 = jnp.concatenate([zero, -ones, y1, zero], axis=1)
    r2 = x2 * p2r2 - p2r0
    r3 = y2 * p2r2 - p2r1
    out_ref[0] = jnp.concatenate([r0, r1, r2, r3], axis=1)


def _build_X4(p1, p2, P2):
    B, N, _ = p1.shape
    flat = pl.pallas_call(
        _x4_body,
        grid=(B,),
        in_specs=[
            pl.BlockSpec((1, N, 2), lambda b: (b, 0, 0)),
            pl.BlockSpec((1, N, 2), lambda b: (b, 0, 0)),
            pl.BlockSpec((1, 3, 4), lambda b: (b, 0, 0)),
        ],
        out_specs=pl.BlockSpec((1, N, 16), lambda b: (b, 0, 0)),
        out_shape=jax.ShapeDtypeStruct((B, N, 16), jnp.float32),
        compiler_params=pltpu.CompilerParams(
            dimension_semantics=("parallel",)),
    )(p1, p2, P2)
    return flat.reshape(B, N, 4, 4)


def _div_body(h_ref, out_ref):
    h = h_ref[0]
    out_ref[0] = h[:, 0:3] / h[:, 3:4]


def _dehomogenize(h):
    B, N, _ = h.shape
    return pl.pallas_call(
        _div_body,
        grid=(B,),
        in_specs=[pl.BlockSpec((1, N, 4), lambda b: (b, 0, 0))],
        out_specs=pl.BlockSpec((1, N, 3), lambda b: (b, 0, 0)),
        out_shape=jax.ShapeDtypeStruct((B, N, 3), jnp.float32),
        compiler_params=pltpu.CompilerParams(
            dimension_semantics=("parallel",)),
    )(h)


def kernel(p1, p2, mask):
    X = _build_X(p1, p2)
    M = jnp.swapaxes(X, -2, -1) @ (mask @ X)
    _, _, Vh9 = jnp.linalg.svd(M)
    F = Vh9[..., :, -1].reshape(-1, 3, 3)
    Ft = jnp.swapaxes(F, -2, -1)
    _, _, Vh3 = jnp.linalg.svd(Ft)
    e2 = Vh3[..., :, -1]
    z = jnp.zeros_like(e2[..., 0])
    cpm = jnp.stack(
        [z, -e2[..., 2], e2[..., 1],
         e2[..., 2], z, -e2[..., 0],
         -e2[..., 1], e2[..., 0], z], axis=-1).reshape(-1, 3, 3)
    R2 = cpm @ F
    P2 = jnp.concatenate([R2, e2[..., None]], axis=-1)
    X4 = _build_X4(p1, p2, P2)
    _, _, Vh4 = jnp.linalg.svd(X4)
    h = Vh4[..., :, -1]
    return _dehomogenize(h)
